# baseline (speedup 1.0000x reference)
import functools

import jax
import jax.numpy as jnp
from jax import lax
from jax.experimental import pallas as pl
from jax.experimental.pallas import tpu as pltpu
from jax.experimental.pallas import tpu_sc as plsc

VOCAB = 100000
EMB = 64
CTX = 20
BATCH = 4096

NUM_CORES = 2
NUM_SUBCORES = 16
NUM_WORKERS = NUM_CORES * NUM_SUBCORES
PB = BATCH // NUM_WORKERS

_sc_mesh = plsc.VectorSubcoreMesh(core_axis_name="c", subcore_axis_name="s")


@functools.partial(
    pl.kernel,
    mesh=_sc_mesh,
    out_type=jax.ShapeDtypeStruct((BATCH, EMB), jnp.float32),
    scratch_types=[
        pltpu.VMEM((CTX, PB), jnp.int32),
        pltpu.VMEM((PB,), jnp.int32),
        pltpu.VMEM((PB, EMB), jnp.float32),
        pltpu.VMEM((PB, EMB), jnp.float32),
        pltpu.VMEM_SHARED((BATCH, EMB), jnp.float32),
        pltpu.SemaphoreType.DMA,
        pltpu.SemaphoreType.DMA,
    ],
    compiler_params=pltpu.CompilerParams(use_tc_tiling_on_sc=False),
)
def _sc_pool(xt_hbm, ids_hbm, table_hbm, out_hbm, idx_v, dst_v, rows_a,
             rows_b, acc_sh, sem_a, sem_b):
    wid = lax.axis_index("s") * NUM_CORES + lax.axis_index("c")
    base = wid * PB
    pltpu.sync_copy(ids_hbm.at[pl.ds(base, PB)], dst_v)
    pltpu.sync_copy(xt_hbm.at[:, pl.ds(base, PB)], idx_v)
    bufs = [(rows_a, sem_a), (rows_b, sem_b)]
    handles = [None] * CTX
    handles[0] = pltpu.async_copy(table_hbm.at[idx_v.at[0]], rows_a, sem_a)
    for j in range(CTX):
        rows, _ = bufs[j % 2]
        if j + 1 < CTX:
            nrows, nsem = bufs[(j + 1) % 2]
            handles[j + 1] = pltpu.async_copy(
                table_hbm.at[idx_v.at[j + 1]], nrows, nsem)
        handles[j].wait()
        pltpu.sync_copy(rows, acc_sh.at[dst_v], add=(j > 0))
    pltpu.sync_copy(acc_sh.at[pl.ds(base, PB)], out_hbm.at[pl.ds(base, PB)])


BB = 4096
VB = 1536


def _mm_body(wt_ref, p_ref, b_ref, o_ref, bcol):
    @pl.when(pl.program_id(1) == 0)
    def _():
        bcol[...] = b_ref[...].T

    o_ref[...] = lax.dot_general(
        wt_ref[...], p_ref[...], (((0,), (1,)), ((), ())),
        preferred_element_type=jnp.float32) + bcol[...]


def _tc_matmul_t(pooled, Wt, brow):
    return pl.pallas_call(
        _mm_body,
        grid=(pl.cdiv(VOCAB, VB), BATCH // BB),
        in_specs=[
            pl.BlockSpec((EMB, VB), lambda i, j: (0, i)),
            pl.BlockSpec((BB, EMB), lambda i, j: (j, 0)),
            pl.BlockSpec((1, VB), lambda i, j: (0, i)),
        ],
        out_specs=pl.BlockSpec((VB, BB), lambda i, j: (i, j)),
        out_shape=jax.ShapeDtypeStruct((VOCAB, BATCH), jnp.float32),
        scratch_shapes=[pltpu.VMEM((VB, 1), jnp.float32)],
        compiler_params=pltpu.CompilerParams(
            dimension_semantics=("parallel", "arbitrary")),
    )(Wt, pooled, brow)


def kernel(x, emb_table, W, b):
    xt = x.T.astype(jnp.int32)
    ids = jnp.arange(BATCH, dtype=jnp.int32)
    pooled = _sc_pool(xt, ids, emb_table) * (1.0 / CTX)
    logits_t = _tc_matmul_t(pooled, W.T, b.reshape(1, VOCAB))
    return logits_t.T


# speedup vs baseline: 1.0130x; 1.0130x over previous
import functools

import jax
import jax.numpy as jnp
from jax import lax
from jax.experimental import pallas as pl
from jax.experimental.pallas import tpu as pltpu
from jax.experimental.pallas import tpu_sc as plsc

VOCAB = 100000
EMB = 64
CTX = 20
BATCH = 4096

NUM_CORES = 2
NUM_SUBCORES = 16
NUM_WORKERS = NUM_CORES * NUM_SUBCORES
PB = BATCH // NUM_WORKERS

_sc_mesh = plsc.VectorSubcoreMesh(core_axis_name="c", subcore_axis_name="s")


@functools.partial(
    pl.kernel,
    mesh=_sc_mesh,
    out_type=jax.ShapeDtypeStruct((BATCH, EMB), jnp.float32),
    scratch_types=[
        pltpu.VMEM((CTX, PB), jnp.int32),
        pltpu.VMEM((PB,), jnp.int32),
        pltpu.VMEM((PB, EMB), jnp.float32),
        pltpu.VMEM((PB, EMB), jnp.float32),
        pltpu.VMEM_SHARED((BATCH, EMB), jnp.float32),
        pltpu.SemaphoreType.DMA,
        pltpu.SemaphoreType.DMA,
    ],
    compiler_params=pltpu.CompilerParams(use_tc_tiling_on_sc=False),
)
def _sc_pool(xt_hbm, ids_hbm, table_hbm, out_hbm, idx_v, dst_v, rows_a,
             rows_b, acc_sh, sem_a, sem_b):
    wid = lax.axis_index("s") * NUM_CORES + lax.axis_index("c")
    base = wid * PB
    pltpu.sync_copy(ids_hbm.at[pl.ds(base, PB)], dst_v)
    pltpu.sync_copy(xt_hbm.at[:, pl.ds(base, PB)], idx_v)
    bufs = [(rows_a, sem_a), (rows_b, sem_b)]
    handles = [None] * CTX
    handles[0] = pltpu.async_copy(table_hbm.at[idx_v.at[0]], rows_a, sem_a)
    for j in range(CTX):
        rows, _ = bufs[j % 2]
        if j + 1 < CTX:
            nrows, nsem = bufs[(j + 1) % 2]
            handles[j + 1] = pltpu.async_copy(
                table_hbm.at[idx_v.at[j + 1]], nrows, nsem)
        handles[j].wait()
        pltpu.sync_copy(rows, acc_sh.at[dst_v], add=(j > 0))
    pltpu.sync_copy(acc_sh.at[pl.ds(base, PB)], out_hbm.at[pl.ds(base, PB)])


BB = 4096
VB = 1024


def _mm_body(wt_ref, p_ref, b_ref, o_ref, bcol):
    @pl.when(pl.program_id(1) == 0)
    def _():
        bcol[...] = b_ref[...].T

    o_ref[...] = lax.dot_general(
        wt_ref[...], p_ref[...], (((0,), (1,)), ((), ())),
        preferred_element_type=jnp.float32) + bcol[...]


def _tc_matmul_t(pooled, Wt, brow):
    return pl.pallas_call(
        _mm_body,
        grid=(pl.cdiv(VOCAB, VB), BATCH // BB),
        in_specs=[
            pl.BlockSpec((EMB, VB), lambda i, j: (0, i)),
            pl.BlockSpec((BB, EMB), lambda i, j: (j, 0)),
            pl.BlockSpec((1, VB), lambda i, j: (0, i)),
        ],
        out_specs=pl.BlockSpec((VB, BB), lambda i, j: (i, j)),
        out_shape=jax.ShapeDtypeStruct((VOCAB, BATCH), jnp.float32),
        scratch_shapes=[pltpu.VMEM((VB, 1), jnp.float32)],
        compiler_params=pltpu.CompilerParams(
            dimension_semantics=("parallel", "arbitrary")),
    )(Wt, pooled, brow)


def kernel(x, emb_table, W, b):
    xt = x.T.astype(jnp.int32)
    ids = jnp.arange(BATCH, dtype=jnp.int32)
    pooled = _sc_pool(xt, ids, emb_table) * (1.0 / CTX)
    logits_t = _tc_matmul_t(pooled, W.T, b.reshape(1, VOCAB))
    return logits_t.T
